# baseline (speedup 1.0000x reference)
import numpy as np

import jax
import jax.numpy as jnp
from jax import lax
from jax.experimental import pallas as pl
from jax.experimental.pallas import tpu as pltpu


def _interp_weights_f32(out_size, in_size):
    scale = (in_size - 1) / (out_size - 1) if out_size > 1 else 0.0
    coords = np.arange(out_size, dtype=np.float32) * np.float32(scale)
    lo = coords.astype(np.int64)
    hi = np.minimum(np.ceil(coords), in_size - 1).astype(np.int64)
    frac = coords - lo.astype(np.float32)
    m = np.zeros((out_size, in_size), dtype=np.float32)
    r = np.arange(out_size)
    m[r, lo] += 1.0 - frac
    m[r, hi] += frac
    return m


def _bilerp_block_kernel(wd_ref, wh_ref, x_ref, o_ref):
    B, H2, W2 = x_ref.shape
    OH = wh_ref.shape[0]
    OW = wd_ref.shape[1] // 2

    xb = x_ref[...].astype(jnp.bfloat16).reshape(B * H2, W2)
    tp = jnp.dot(xb, wd_ref[...], preferred_element_type=jnp.float32)
    tp = tp.astype(jnp.bfloat16).reshape(B, H2, 2 * OW)

    tcat = jnp.concatenate([tp[:, :, :OW], tp[:, :, OW:]], axis=1)

    wh_b = jnp.broadcast_to(wh_ref[...], (B, OH, 2 * H2))
    o = lax.dot_general(
        wh_b,
        tcat,
        dimension_numbers=(((2,), (1,)), ((0,), (0,))),
        preferred_element_type=jnp.float32,
    )
    o_ref[...] = o


def kernel(x):
    N, C, H, W = x.shape
    OH, OW = 128, 128
    NC = N * C
    B = 128
    assert NC % B == 0 and H % 16 == 0
    steps = NC // B

    ww_t = np.ascontiguousarray(_interp_weights_f32(OW, W).T)
    wd = np.zeros((2 * W, 2 * OW), dtype=np.float32)
    wd[:W, :OW] = ww_t
    wd[W:, OW:] = ww_t

    perm = np.concatenate([np.arange(0, H, 2), np.arange(1, H, 2)])
    whp = _interp_weights_f32(OH, H)[:, perm]

    wd = jnp.asarray(wd, dtype=jnp.bfloat16)
    whp = jnp.asarray(whp, dtype=jnp.bfloat16)
    x3 = x.reshape(NC, H // 2, 2 * W)

    out = pl.pallas_call(
        _bilerp_block_kernel,
        out_shape=jax.ShapeDtypeStruct((NC, OH, OW), jnp.float32),
        grid=(steps,),
        in_specs=[
            pl.BlockSpec((2 * W, 2 * OW), lambda i: (0, 0)),
            pl.BlockSpec((OH, H), lambda i: (0, 0)),
            pl.BlockSpec((B, H // 2, 2 * W), lambda i: (i, 0, 0)),
        ],
        out_specs=pl.BlockSpec((B, OH, OW), lambda i: (i, 0, 0)),
        compiler_params=pltpu.CompilerParams(
            dimension_semantics=("parallel",),
        ),
    )(wd, whp, x3)
    return out.reshape(N, C, OH, OW)


# speedup vs baseline: 1.6806x; 1.6806x over previous
import numpy as np

import jax
import jax.numpy as jnp
from jax import lax
from jax.experimental import pallas as pl
from jax.experimental.pallas import tpu as pltpu


def _interp_weights_f32(out_size, in_size):
    scale = (in_size - 1) / (out_size - 1) if out_size > 1 else 0.0
    coords = np.arange(out_size, dtype=np.float32) * np.float32(scale)
    lo = coords.astype(np.int64)
    hi = np.minimum(np.ceil(coords), in_size - 1).astype(np.int64)
    frac = coords - lo.astype(np.float32)
    m = np.zeros((out_size, in_size), dtype=np.float32)
    r = np.arange(out_size)
    m[r, lo] += 1.0 - frac
    m[r, hi] += frac
    return m


def _bilerp_block_kernel(ww_ref, wh_ref, x_ref, o_ref):
    B, H, W = x_ref.shape
    OH = wh_ref.shape[0]
    OW = ww_ref.shape[1]

    xb = x_ref[...].astype(jnp.bfloat16).reshape(B * H, W)
    t = jnp.dot(xb, ww_ref[...], preferred_element_type=jnp.float32)
    tb = t.astype(jnp.bfloat16).reshape(B, H, OW)

    wh_b = jnp.broadcast_to(wh_ref[...], (B, OH, H))
    o = lax.dot_general(
        wh_b,
        tb,
        dimension_numbers=(((2,), (1,)), ((0,), (0,))),
        preferred_element_type=jnp.float32,
    )
    o_ref[...] = o


def kernel(x):
    N, C, H, W = x.shape
    OH, OW = 128, 128
    NC = N * C
    B = 256
    assert NC % B == 0
    steps = NC // B

    wh = jnp.asarray(_interp_weights_f32(OH, H), dtype=jnp.bfloat16)
    wwt = jnp.asarray(
        np.ascontiguousarray(_interp_weights_f32(OW, W).T), dtype=jnp.bfloat16
    )
    x3 = x.reshape(NC, H, W)

    out = pl.pallas_call(
        _bilerp_block_kernel,
        out_shape=jax.ShapeDtypeStruct((NC, OH, OW), jnp.float32),
        grid=(steps,),
        in_specs=[
            pl.BlockSpec((W, OW), lambda i: (0, 0)),
            pl.BlockSpec((OH, H), lambda i: (0, 0)),
            pl.BlockSpec((B, H, W), lambda i: (i, 0, 0)),
        ],
        out_specs=pl.BlockSpec((B, OH, OW), lambda i: (i, 0, 0)),
        compiler_params=pltpu.CompilerParams(
            dimension_semantics=("parallel",),
        ),
    )(wwt, wh, x3)
    return out.reshape(N, C, OH, OW)
